# baseline (speedup 1.0000x reference)
import functools

import jax
import jax.numpy as jnp
from jax import lax
from jax.experimental import pallas as pl
from jax.experimental.pallas import tpu as pltpu
from jax.experimental.pallas import tpu_sc as plsc

D_MODEL = 2048
SEQ_LEN = 2048

NUM_CORES = 2
NUM_SUBCORES = 16
NUM_WORKERS = NUM_CORES * NUM_SUBCORES
ROWS_PER_WORKER = SEQ_LEN // NUM_WORKERS
CHUNK_ROWS = 16
NUM_CHUNKS = ROWS_PER_WORKER // CHUNK_ROWS

_mesh = plsc.VectorSubcoreMesh(
    core_axis_name="c", subcore_axis_name="s",
    num_cores=NUM_CORES, num_subcores=NUM_SUBCORES,
)


@functools.partial(
    pl.kernel,
    mesh=_mesh,
    out_type=jax.ShapeDtypeStruct((SEQ_LEN, D_MODEL), jnp.float32),
    scratch_types=[
        pltpu.VMEM((CHUNK_ROWS, D_MODEL), jnp.float32),
        pltpu.VMEM((CHUNK_ROWS, D_MODEL), jnp.float32),
        pltpu.SemaphoreType.DMA,
        pltpu.SemaphoreType.DMA,
        pltpu.SemaphoreType.DMA,
        pltpu.SemaphoreType.DMA,
    ],
)
def _gather_rows(table_hbm, out_hbm, buf0, buf1, isem0, isem1, osem0, osem1):
    wid = lax.axis_index("s") * NUM_CORES + lax.axis_index("c")
    base = wid * ROWS_PER_WORKER
    bufs = (buf0, buf1)
    isems = (isem0, isem1)
    osems = (osem0, osem1)

    def copy_in(i):
        b = i % 2
        return pltpu.make_async_copy(
            table_hbm.at[pl.ds(base + i * CHUNK_ROWS, CHUNK_ROWS)],
            bufs[b], isems[b])

    def copy_out(i):
        b = i % 2
        return pltpu.make_async_copy(
            bufs[b], out_hbm.at[pl.ds(base + i * CHUNK_ROWS, CHUNK_ROWS)],
            osems[b])

    copy_in(0).start()
    for i in range(NUM_CHUNKS):
        copy_in(i).wait()
        copy_out(i).start()
        if i + 1 < NUM_CHUNKS:
            if i >= 1:
                copy_out(i - 1).wait()
            copy_in(i + 1).start()
    copy_out(NUM_CHUNKS - 2).wait()
    copy_out(NUM_CHUNKS - 1).wait()


def kernel(input_ids, pos_embedding):
    del input_ids
    out = _gather_rows(pos_embedding)
    return out[None]
